# baseline (speedup 1.0000x reference)
import functools

import jax
import jax.numpy as jnp
from jax.experimental import pallas as pl
from jax.experimental.pallas import tpu as pltpu
from jax.experimental.pallas import tpu_sc as plsc


def _booth_round(x):
    xi = jax.lax.bitcast_convert_type(x, jnp.uint32)
    ri = (xi + jnp.uint32(0x1FFFFF) + ((xi >> jnp.uint32(22)) & jnp.uint32(1))) & jnp.uint32(0xFFC00000)
    r = jax.lax.bitcast_convert_type(ri, jnp.float32)
    r = jnp.minimum(jnp.maximum(r, -1.0), 1.0)
    a = jnp.abs(x)
    sval = jax.lax.bitcast_convert_type(
        (xi & jnp.uint32(0x80000000)) | jnp.uint32(0x3BC00000), jnp.float32
    )
    return jnp.where(
        a <= 0.0029296875, 0.0, jnp.where(a <= 0.0048828125, sval, r)
    )


_CC = 1
_NCHW = 24
_NBUF = 6


def kernel(x, booth_values):
    del booth_values
    B, C, W, H = x.shape
    rows = _CC * W
    mesh = plsc.VectorSubcoreMesh(core_axis_name="core", subcore_axis_name="subcore")

    @functools.partial(
        pl.kernel,
        out_type=jax.ShapeDtypeStruct((B, C, W, H), jnp.float32),
        mesh=mesh,
        scratch_types=[
            pltpu.VMEM((_NBUF, rows, H), jnp.float32),
            pltpu.VMEM((_NBUF, rows, H), jnp.float32),
            pltpu.SemaphoreType.DMA((_NBUF,)),
            pltpu.SemaphoreType.DMA((_NBUF,)),
        ],
    )
    def sc_quant(x_hbm, o_hbm, in_b, out_b, in_sems, out_sems):
        from jax import lax

        wid = lax.axis_index("subcore") * 2 + lax.axis_index("core")
        b = wid // 8
        c0 = (wid % 8) * (_NCHW * _CC)

        def in_copy(i, s):
            src = x_hbm.at[b, pl.ds(c0 + i * _CC, _CC)].reshape(rows, H)
            return pltpu.make_async_copy(src, in_b.at[s], in_sems.at[s])

        def out_copy(i, s):
            dst = o_hbm.at[b, pl.ds(c0 + i * _CC, _CC)].reshape(rows, H)
            return pltpu.make_async_copy(out_b.at[s], dst, out_sems.at[s])

        for i in range(_NBUF):
            in_copy(i, i).start()
        for i in range(_NCHW):
            s = i % _NBUF
            in_copy(i, s).wait()
            if i >= _NBUF:
                out_copy(i - _NBUF, s).wait()

            @pl.loop(0, rows, step=4)
            def _(r):
                for dr in range(4):
                    for o in (0, 16, 32, 40):
                        sl = pl.ds(o, 16)
                        out_b.at[s, r + dr, sl][...] = _booth_round(
                            in_b.at[s, r + dr, sl][...]
                        )

            out_copy(i, s).start()
            if i + _NBUF < _NCHW:
                in_copy(i + _NBUF, s).start()
        for i in range(_NCHW - _NBUF, _NCHW):
            out_copy(i, i % _NBUF).wait()

    return sc_quant(x)


# speedup vs baseline: 1.0331x; 1.0331x over previous
import functools

import jax
import jax.numpy as jnp
from jax.experimental import pallas as pl
from jax.experimental.pallas import tpu as pltpu
from jax.experimental.pallas import tpu_sc as plsc


def _booth_round(x):
    xi = jax.lax.bitcast_convert_type(x, jnp.uint32)
    ri = (xi + jnp.uint32(0x1FFFFF) + ((xi >> jnp.uint32(22)) & jnp.uint32(1))) & jnp.uint32(0xFFC00000)
    r = jax.lax.bitcast_convert_type(ri, jnp.float32)
    r = jnp.minimum(jnp.maximum(r, -1.0), 1.0)
    a = jnp.abs(x)
    sval = jax.lax.bitcast_convert_type(
        (xi & jnp.uint32(0x80000000)) | jnp.uint32(0x3BC00000), jnp.float32
    )
    return jnp.where(
        a <= 0.0029296875, 0.0, jnp.where(a <= 0.0048828125, sval, r)
    )


_CC = 2
_NCHW = 12
_NBUF = 4


def kernel(x, booth_values):
    del booth_values
    B, C, W, H = x.shape
    rows = _CC * W
    mesh = plsc.VectorSubcoreMesh(core_axis_name="core", subcore_axis_name="subcore")

    @functools.partial(
        pl.kernel,
        out_type=jax.ShapeDtypeStruct((B, C, W, H), jnp.float32),
        mesh=mesh,
        scratch_types=[
            pltpu.VMEM((_NBUF, rows, H), jnp.float32),
            pltpu.VMEM((_NBUF, rows, H), jnp.float32),
            pltpu.SemaphoreType.DMA((_NBUF,)),
            pltpu.SemaphoreType.DMA((_NBUF,)),
        ],
    )
    def sc_quant(x_hbm, o_hbm, in_b, out_b, in_sems, out_sems):
        from jax import lax

        wid = lax.axis_index("subcore") * 2 + lax.axis_index("core")
        b = wid // 8
        c0 = (wid % 8) * (_NCHW * _CC)

        def in_copy(i, s):
            src = x_hbm.at[b, pl.ds(c0 + i * _CC, _CC)].reshape(rows, H)
            return pltpu.make_async_copy(src, in_b.at[s], in_sems.at[s])

        def out_copy(i, s):
            dst = o_hbm.at[b, pl.ds(c0 + i * _CC, _CC)].reshape(rows, H)
            return pltpu.make_async_copy(out_b.at[s], dst, out_sems.at[s])

        for i in range(_NBUF):
            in_copy(i, i).start()
        for i in range(_NCHW):
            s = i % _NBUF
            in_copy(i, s).wait()
            if i >= _NBUF:
                out_copy(i - _NBUF, s).wait()

            @pl.loop(0, rows)
            def _(r):
                for o in (0, 16, 32, 40):
                    sl = pl.ds(o, 16)
                    out_b.at[s, r, sl][...] = _booth_round(in_b.at[s, r, sl][...])

            out_copy(i, s).start()
            if i + _NBUF < _NCHW:
                in_copy(i + _NBUF, s).start()
        for i in range(_NCHW - _NBUF, _NCHW):
            out_copy(i, i % _NBUF).wait()

    return sc_quant(x)
